# baseline (speedup 1.0000x reference)
import functools

import jax
import jax.numpy as jnp
from jax import lax
from jax.experimental import pallas as pl
from jax.experimental.pallas import tpu as pltpu
from jax.experimental.pallas import tpu_sc as plsc

_BATCH = 16384
_HIST = 50
_DIM = 32
_NUM_IDX = _BATCH * _HIST
_NUM_WORKERS = 32
_PER_WORKER = _NUM_IDX // _NUM_WORKERS
_CHUNK = 1280
_NUM_CHUNKS = _PER_WORKER // _CHUNK
_NBUF = 3


def kernel(x, table):
    idx = x.reshape(_NUM_IDX).astype(jnp.int32)
    mesh = plsc.VectorSubcoreMesh(core_axis_name="c", subcore_axis_name="s")

    @functools.partial(
        pl.kernel,
        mesh=mesh,
        out_type=jax.ShapeDtypeStruct((_NUM_IDX, _DIM), jnp.float32),
        compiler_params=pltpu.CompilerParams(use_tc_tiling_on_sc=False),
        scratch_types=(
            [pltpu.VMEM((_CHUNK,), jnp.int32)] * _NBUF
            + [pltpu.VMEM((_CHUNK, _DIM), jnp.float32)] * _NBUF
            + [pltpu.SemaphoreType.DMA] * (2 * _NBUF)
        ),
    )
    def gather_kernel(table_hbm, idx_hbm, out_hbm, *scratch):
        idx_v = scratch[:_NBUF]
        rows = scratch[_NBUF : 2 * _NBUF]
        gsem = scratch[2 * _NBUF : 3 * _NBUF]
        osem = scratch[3 * _NBUF :]
        wid = lax.axis_index("s") * 2 + lax.axis_index("c")
        base = wid * _PER_WORKER

        gather_h = [None] * _NBUF
        out_h = [None] * _NBUF

        def retire(d):
            dbuf = d % _NBUF
            gather_h[dbuf].wait()
            out_h[dbuf] = pltpu.async_copy(
                rows[dbuf],
                out_hbm.at[pl.ds(base + d * _CHUNK, _CHUNK)],
                osem[dbuf],
            )

        for c in range(_NUM_CHUNKS):
            buf = c % _NBUF
            if out_h[buf] is not None:
                out_h[buf].wait()
            pltpu.sync_copy(
                idx_hbm.at[pl.ds(base + c * _CHUNK, _CHUNK)], idx_v[buf]
            )
            gather_h[buf] = pltpu.async_copy(
                table_hbm.at[idx_v[buf]], rows[buf], gsem[buf]
            )
            if c >= _NBUF - 1:
                retire(c - (_NBUF - 1))

        for d in range(_NUM_CHUNKS - (_NBUF - 1), _NUM_CHUNKS):
            retire(d)
        for h in out_h:
            if h is not None:
                h.wait()

    out = gather_kernel(table, idx)
    return out.reshape(_BATCH, _HIST, _DIM)


# speedup vs baseline: 1.7300x; 1.7300x over previous
import functools

import jax
import jax.numpy as jnp
from jax import lax
from jax.experimental import pallas as pl
from jax.experimental.pallas import tpu as pltpu
from jax.experimental.pallas import tpu_sc as plsc

_BATCH = 16384
_HIST = 50
_DIM = 32
_VOCAB = 1000000
_NUM_IDX = _BATCH * _HIST
_NUM_WORKERS = 32
_CHUNK = 1024
_BLOCKS_PER_T = _BATCH // _CHUNK
_NUM_UNITS = _HIST * _BLOCKS_PER_T
_UNITS_PER_WORKER = _NUM_UNITS // _NUM_WORKERS
_NBUF = 3


def kernel(x, table):
    xt = x.T
    rt = table[:_VOCAB, :].reshape(_VOCAB * _DIM).reshape(_VOCAB, _DIM)
    mesh = plsc.VectorSubcoreMesh(core_axis_name="c", subcore_axis_name="s")

    @functools.partial(
        pl.kernel,
        mesh=mesh,
        out_type=jax.ShapeDtypeStruct((_NUM_IDX, _DIM), jnp.float32),
        compiler_params=pltpu.CompilerParams(use_tc_tiling_on_sc=False),
        scratch_types=(
            [pltpu.VMEM((_CHUNK,), jnp.int32)] * _NBUF
            + [pltpu.VMEM((_CHUNK, _DIM), jnp.float32)] * _NBUF
            + [pltpu.SemaphoreType.DMA] * (2 * _NBUF)
        ),
    )
    def gather_kernel(table_hbm, idx_hbm, out_hbm, *scratch):
        idx_v = scratch[:_NBUF]
        rows = scratch[_NBUF : 2 * _NBUF]
        gsem = scratch[2 * _NBUF : 3 * _NBUF]
        osem = scratch[3 * _NBUF :]
        wid = lax.axis_index("s") * 2 + lax.axis_index("c")
        u0 = wid * _UNITS_PER_WORKER

        gather_h = [None] * _NBUF
        out_h = [None] * _NBUF
        offs = [None] * _NBUF

        def retire(k):
            kbuf = k % _NBUF
            gather_h[kbuf].wait()
            out_h[kbuf] = pltpu.async_copy(
                rows[kbuf],
                out_hbm.at[pl.ds(offs[kbuf], _CHUNK)],
                osem[kbuf],
            )

        for k in range(_UNITS_PER_WORKER):
            buf = k % _NBUF
            u = u0 + k
            t = u // _BLOCKS_PER_T
            b0 = (u % _BLOCKS_PER_T) * _CHUNK
            if out_h[buf] is not None:
                out_h[buf].wait()
            offs[buf] = t * _BATCH + b0
            pltpu.sync_copy(idx_hbm.at[t, pl.ds(b0, _CHUNK)], idx_v[buf])
            gather_h[buf] = pltpu.async_copy(
                table_hbm.at[idx_v[buf]], rows[buf], gsem[buf]
            )
            if k >= _NBUF - 1:
                retire(k - (_NBUF - 1))

        for k in range(_UNITS_PER_WORKER - (_NBUF - 1), _UNITS_PER_WORKER):
            retire(k)
        for h in out_h:
            if h is not None:
                h.wait()

    out2d = gather_kernel(rt, xt)
    return out2d.reshape(_HIST, _BATCH, _DIM).transpose(1, 0, 2)
